# baseline (speedup 1.0000x reference)
import functools

import jax
import jax.numpy as jnp
from jax import lax
from jax.experimental import pallas as pl
from jax.experimental.pallas import tpu as pltpu
from jax.experimental.pallas import tpu_sc as plsc

B = 4096
L = 200
DIM = 64
LANES = 16

_info = plsc.get_sparse_core_info()
NC = _info.num_cores
NS = _info.num_subcores
NW = NC * NS
EPW = B // NW
TOK = EPW * L
C1 = 104
C2 = 96
GROUPS = EPW // LANES

_mesh = plsc.VectorSubcoreMesh(core_axis_name="c", subcore_axis_name="s")

_DNUMS = lax.GatherDimensionNumbers(
    offset_dims=(), collapsed_slice_dims=(0,), start_index_map=(0,))


def _permute(x, idx):
    return lax.gather(x, idx[:, None], _DNUMS, (1,),
                      mode=lax.GatherScatterMode.PROMISE_IN_BOUNDS)


@functools.partial(
    pl.kernel,
    mesh=_mesh,
    compiler_params=pltpu.CompilerParams(use_tc_tiling_on_sc=False),
    out_type=jax.ShapeDtypeStruct((B,), jnp.float32),
    scratch_types=[
        pltpu.VMEM((TOK,), jnp.int32),
        pltpu.VMEM((EPW,), jnp.int32),
        pltpu.VMEM((DIM,), jnp.float32),
        pltpu.VMEM((LANES,), jnp.float32),
        pltpu.VMEM((C1, DIM), jnp.float32),
        pltpu.VMEM((C2, DIM), jnp.float32),
        pltpu.VMEM((EPW,), jnp.float32),
        pltpu.SemaphoreType.DMA,
        pltpu.SemaphoreType.DMA,
    ],
)
def _gather_pool(tflat_hbm, lens_hbm, table_hbm, w_hbm, b_hbm, out_hbm,
                 tidx, len_v, w_v, b_v, rb1, rb2, out_v, sem1, sem2):
    wid = lax.axis_index("s") * NC + lax.axis_index("c")
    base = pl.multiple_of(wid * EPW, 8)

    pltpu.sync_copy(tflat_hbm.at[pl.ds(base * L, TOK)], tidx)
    pltpu.sync_copy(lens_hbm.at[pl.ds(base, EPW)], len_v)
    pltpu.sync_copy(w_hbm, w_v)
    pltpu.sync_copy(b_hbm, b_v)

    w0 = w_v[pl.ds(0, LANES)]
    w1 = w_v[pl.ds(LANES, LANES)]
    w2 = w_v[pl.ds(2 * LANES, LANES)]
    w3 = w_v[pl.ds(3 * LANES, LANES)]
    bv = b_v[...]
    lane = lax.broadcasted_iota(jnp.int32, (LANES,), 0)

    def accum8(ref, r0, accs):
        a0, a1, a2, a3 = accs
        for u in range(8):
            r = r0 + u
            a0 = a0 + ref[r, pl.ds(0, LANES)]
            a1 = a1 + ref[r, pl.ds(LANES, LANES)]
            a2 = a2 + ref[r, pl.ds(2 * LANES, LANES)]
            a3 = a3 + ref[r, pl.ds(3 * LANES, LANES)]
        return a0, a1, a2, a3

    def group_body(g, carry):
        def ex_body(e16, cur):
            e = g * LANES + e16
            off = pl.multiple_of(e * L, 8)
            cp1 = pltpu.async_copy(
                table_hbm.at[tidx.at[pl.ds(off, C1)]], rb1, sem1)
            cp2 = pltpu.async_copy(
                table_hbm.at[tidx.at[pl.ds(off + C1, C2)]], rb2, sem2)
            cp1.wait()
            cp2.wait()

            zero = jnp.zeros((LANES,), jnp.float32)
            accs = (zero, zero, zero, zero)
            accs = lax.fori_loop(
                0, C1 // 8, lambda i, a: accum8(rb1, i * 8, a), accs)
            accs = lax.fori_loop(
                0, C2 // 8, lambda i, a: accum8(rb2, i * 8, a), accs)
            a0, a1, a2, a3 = accs
            part = a0 * w0 + a1 * w1 + a2 * w2 + a3 * w3
            for sh in (8, 4, 2, 1):
                part = part + _permute(part, lane ^ sh)
            return jnp.where(lane == e16, part, cur)

        cur = lax.fori_loop(0, LANES, ex_body,
                            jnp.zeros((LANES,), jnp.float32))
        lg = len_v[pl.ds(g * LANES, LANES)].astype(jnp.float32)
        out_v[pl.ds(g * LANES, LANES)] = cur / lg + bv
        return carry

    lax.fori_loop(0, GROUPS, group_body, 0)
    pltpu.sync_copy(out_v, out_hbm.at[pl.ds(base, EPW)])


def kernel(text, text_lengths, table, W, b):
    mask = jnp.arange(L, dtype=jnp.int32)[None, :] < text_lengths[:, None].astype(jnp.int32)
    tflat = jnp.where(mask, text.astype(jnp.int32), 0).reshape(-1)
    lens = text_lengths.astype(jnp.int32)
    w64 = W.reshape(DIM).astype(jnp.float32)
    b16 = jnp.broadcast_to(b.astype(jnp.float32), (LANES,))
    out = _gather_pool(tflat, lens, table, w64, b16)
    return out.reshape(B, 1)


# speedup vs baseline: 11.9947x; 11.9947x over previous
import functools

import jax
import jax.numpy as jnp
from jax import lax
from jax.experimental import pallas as pl
from jax.experimental.pallas import tpu as pltpu
from jax.experimental.pallas import tpu_sc as plsc

B = 4096
L = 200
DIM = 64
LANES = 16

_info = plsc.get_sparse_core_info()
NC = _info.num_cores
NS = _info.num_subcores
NW = NC * NS
EPW = B // NW
TOK = EPW * L
C1 = 104
C2 = 96
NBUF = 4

_mesh = plsc.VectorSubcoreMesh(core_axis_name="c", subcore_axis_name="s")

_DNUMS = lax.GatherDimensionNumbers(
    offset_dims=(), collapsed_slice_dims=(0,), start_index_map=(0,))


def _permute(x, idx):
    return lax.gather(x, idx[:, None], _DNUMS, (1,),
                      mode=lax.GatherScatterMode.PROMISE_IN_BOUNDS)


@functools.partial(
    pl.kernel,
    mesh=_mesh,
    compiler_params=pltpu.CompilerParams(use_tc_tiling_on_sc=False),
    out_type=jax.ShapeDtypeStruct((B,), jnp.float32),
    scratch_types=[
        pltpu.VMEM((TOK,), jnp.int32),
        pltpu.VMEM((EPW,), jnp.int32),
        pltpu.VMEM((DIM,), jnp.float32),
        pltpu.VMEM((LANES,), jnp.float32),
        pltpu.VMEM((L, DIM), jnp.float32),
        pltpu.VMEM((L, DIM), jnp.float32),
        pltpu.VMEM((L, DIM), jnp.float32),
        pltpu.VMEM((L, DIM), jnp.float32),
        pltpu.VMEM((EPW,), jnp.float32),
        pltpu.SemaphoreType.DMA,
        pltpu.SemaphoreType.DMA,
        pltpu.SemaphoreType.DMA,
        pltpu.SemaphoreType.DMA,
    ],
)
def _gather_pool(tflat_hbm, lens_hbm, table_hbm, w_hbm, b_hbm, out_hbm,
                 tidx, len_v, w_v, b_v, rb0, rb1, rb2, rb3, out_v,
                 sem0, sem1, sem2, sem3):
    wid = lax.axis_index("s") * NC + lax.axis_index("c")
    base = pl.multiple_of(wid * EPW, 8)
    rbs = (rb0, rb1, rb2, rb3)
    sems = (sem0, sem1, sem2, sem3)

    pltpu.sync_copy(tflat_hbm.at[pl.ds(base * L, TOK)], tidx)
    pltpu.sync_copy(lens_hbm.at[pl.ds(base, EPW)], len_v)
    pltpu.sync_copy(w_hbm, w_v)
    pltpu.sync_copy(b_hbm, b_v)

    w0 = w_v[pl.ds(0, LANES)]
    w1 = w_v[pl.ds(LANES, LANES)]
    w2 = w_v[pl.ds(2 * LANES, LANES)]
    w3 = w_v[pl.ds(3 * LANES, LANES)]
    bv = b_v[...]
    lane = lax.broadcasted_iota(jnp.int32, (LANES,), 0)

    def issue(e, rb, sem):
        off = pl.multiple_of(e * L, 8)
        pltpu.async_copy(
            table_hbm.at[tidx.at[pl.ds(off, C1)]], rb.at[pl.ds(0, C1)], sem)
        pltpu.async_copy(
            table_hbm.at[tidx.at[pl.ds(off + C1, C2)]],
            rb.at[pl.ds(C1, C2)], sem)

    def drain(rb, sem):
        pltpu.make_async_copy(table_hbm.at[pl.ds(0, L)], rb, sem).wait()

    def addrow(rb, r, accs):
        a0, a1, a2, a3 = accs
        a0 = a0 + rb[r, pl.ds(0, LANES)]
        a1 = a1 + rb[r, pl.ds(LANES, LANES)]
        a2 = a2 + rb[r, pl.ds(2 * LANES, LANES)]
        a3 = a3 + rb[r, pl.ds(3 * LANES, LANES)]
        return a0, a1, a2, a3

    def process(e, rb, cur):
        e16 = e % LANES
        g0 = pl.multiple_of(e - e16, LANES)
        lv = len_v[pl.ds(g0, LANES)]
        ln = _permute(lv, (lane + e16) % LANES)[0]
        n8 = ln // 8
        zero = jnp.zeros((LANES,), jnp.float32)
        accs = (zero, zero, zero, zero)

        def acc8(i, a):
            r0 = i * 8
            for u in range(8):
                a = addrow(rb, r0 + u, a)
            return a

        accs = lax.fori_loop(0, n8, acc8, accs)
        accs = lax.fori_loop(n8 * 8, ln,
                             lambda r, a: addrow(rb, r, a), accs)
        a0, a1, a2, a3 = accs
        part = a0 * w0 + a1 * w1 + a2 * w2 + a3 * w3
        for sh in (8, 4, 2, 1):
            part = part + _permute(part, lane ^ sh)
        cur = jnp.where(lane == e16, part, cur)

        @pl.when(e16 == LANES - 1)
        def _():
            lg = len_v[pl.ds(g0, LANES)].astype(jnp.float32)
            out_v[pl.ds(g0, LANES)] = cur / lg + bv

        return cur

    for u in range(NBUF):
        issue(u, rbs[u], sems[u])

    def body(i, cur):
        for u in range(NBUF):
            e = i * NBUF + u
            drain(rbs[u], sems[u])
            cur = process(e, rbs[u], cur)

            @pl.when(e + NBUF < EPW)
            def _():
                issue(e + NBUF, rbs[u], sems[u])
        return cur

    lax.fori_loop(0, EPW // NBUF, body, jnp.zeros((LANES,), jnp.float32))
    pltpu.sync_copy(out_v, out_hbm.at[pl.ds(base, EPW)])


def kernel(text, text_lengths, table, W, b):
    tflat = text.astype(jnp.int32).reshape(-1)
    lens = text_lengths.astype(jnp.int32)
    w64 = W.reshape(DIM).astype(jnp.float32)
    b16 = jnp.broadcast_to(b.astype(jnp.float32), (LANES,))
    out = _gather_pool(tflat, lens, table, w64, b16)
    return out.reshape(B, 1)
